# Initial kernel scaffold; baseline (speedup 1.0000x reference)
import jax
import jax.numpy as jnp
from jax.experimental import pallas as pl


def kernel(preds, target):
    raise NotImplementedError("write your pallas kernel here")


# speedup vs baseline: 128.8588x; 128.8588x over previous
import functools

import jax
import jax.numpy as jnp
from jax import lax
from jax.experimental import pallas as pl
from jax.experimental.pallas import tpu as pltpu
from jax.experimental.pallas import tpu_sc as plsc

C = 21
HW = 512 * 512
B = 4
K = 1024
HSIZE = 2 * C * K
NTEC = 32
PER_TEC = HW // NTEC
S = 1024
CPB = PER_TEC // S
NCHUNK = B * CPB
FGOFF = C * K


def _sc_body(preds_hbm, labels_hbm, out_hbm, buf, lbuf, hist, sem0, sem1):
    wid = lax.axis_index("c") * 16 + lax.axis_index("s")
    sems = (sem0, sem1)

    def zero_body(i, _):
        hist[pl.ds(i * 16, 16)] = jnp.zeros((16,), jnp.int32)
        return 0

    lax.fori_loop(0, HSIZE // 16, zero_body, 0)

    def issue(t, slot):
        b = t // CPB
        start = wid * PER_TEC + (t % CPB) * S
        for c in range(C):
            pltpu.async_copy(
                preds_hbm.at[pl.ds((b * C + c) * HW + start, S)],
                buf.at[pl.ds((slot * C + c) * S, S)],
                sems[slot],
            )
        pltpu.async_copy(
            labels_hbm.at[pl.ds(b * HW + start, S)],
            lbuf.at[pl.ds(slot * S, S)],
            sems[slot],
        )

    def drain(slot):
        for c in range(C):
            pltpu.make_async_copy(
                preds_hbm.at[pl.ds(0, S)],
                buf.at[pl.ds((slot * C + c) * S, S)],
                sems[slot],
            ).wait()
        pltpu.make_async_copy(
            labels_hbm.at[pl.ds(0, S)], lbuf.at[pl.ds(slot * S, S)], sems[slot]
        ).wait()

    def compute(slot):
        def group(i, _):
            s0 = i * 16
            es = []
            tot = jnp.zeros((16,), jnp.float32)
            for c in range(C):
                ex = jnp.exp(buf[pl.ds((slot * C + c) * S + s0, 16)])
                es.append(ex)
                tot = tot + ex
            inv = jnp.float32(1.0) / tot
            lbl = lbuf[pl.ds(slot * S + s0, 16)]
            for c in range(C):
                p = es[c] * inv
                fgb = lbl == c
                e = jnp.where(fgb, jnp.float32(1.0) - p, p)
                t = e * jnp.float32(K)
                bi = jnp.minimum(t.astype(jnp.int32), K - 1) + c * K
                idx = bi + jnp.where(fgb, FGOFF, 0)
                plsc.addupdate_scatter(hist, [idx], jnp.ones((16,), jnp.int32))
            return 0

        lax.fori_loop(0, S // 16, group, 0)

    issue(jnp.int32(0), 0)

    def chunk_pair(j, _):
        t0 = j * 2
        issue(t0 + 1, 1)
        drain(0)
        compute(0)

        @pl.when(t0 + 2 < NCHUNK)
        def _():
            issue(t0 + 2, 0)

        drain(1)
        compute(1)
        return 0

    lax.fori_loop(0, NCHUNK // 2, chunk_pair, 0)

    pltpu.sync_copy(hist, out_hbm.at[pl.ds(wid * HSIZE, HSIZE)])


@functools.cache
def _sc_hist():
    return functools.partial(
        pl.kernel,
        out_type=jax.ShapeDtypeStruct((NTEC * HSIZE,), jnp.int32),
        mesh=plsc.VectorSubcoreMesh(core_axis_name="c", subcore_axis_name="s"),
        compiler_params=pltpu.CompilerParams(needs_layout_passes=False),
        scratch_types=[
            pltpu.VMEM((2 * C * S,), jnp.float32),
            pltpu.VMEM((2 * S,), jnp.int32),
            pltpu.VMEM((HSIZE,), jnp.int32),
            pltpu.SemaphoreType.DMA,
            pltpu.SemaphoreType.DMA,
        ],
    )(_sc_body)


def _tc_body(hist_ref, out_ref):
    h = hist_ref[...].astype(jnp.float32)
    acc = jnp.sum(h, axis=0)
    hb = acc[0:C, :]
    hf = acc[C : 2 * C, :]

    row = lax.broadcasted_iota(jnp.int32, (K, K), 0)
    col = lax.broadcasted_iota(jnp.int32, (K, K), 1)
    ge = (row >= col).astype(jnp.float32)
    f_incl = jax.lax.dot(hf, ge, preferred_element_type=jnp.float32)
    b_incl = jax.lax.dot(hb, ge, preferred_element_type=jnp.float32)
    f_excl = f_incl - hf
    b_excl = b_incl - hb

    g = jnp.sum(hf, axis=1, keepdims=True)

    def jac(f, bb):
        return 1.0 - (g - f) / jnp.maximum(g + bb, 1.0)

    dj = jac(f_incl, b_incl) - jac(f_excl, b_excl)
    ebar = (lax.broadcasted_iota(jnp.int32, (1, K), 1).astype(jnp.float32) + 0.5) * jnp.float32(1.0 / K)
    loss_c = jnp.sum(ebar * dj, axis=1, keepdims=True)
    present = (g > 0).astype(jnp.float32)
    num = jnp.sum(loss_c * present)
    den = jnp.maximum(jnp.sum(present), 1.0)
    out_ref[...] = jnp.broadcast_to(num / den, (1, 1))


def _tc_finish(hist3):
    return pl.pallas_call(
        _tc_body,
        out_shape=jax.ShapeDtypeStruct((1, 1), jnp.float32),
    )(hist3)


def kernel(preds, target):
    x = preds[0].reshape(B * C * HW)
    lbl = target.reshape(B * HW)
    hist = _sc_hist()(x, lbl)
    hist3 = hist.reshape(NTEC, 2 * C, K)
    out = _tc_finish(hist3)
    return out[0, 0]


# speedup vs baseline: 172.3211x; 1.3373x over previous
import functools

import jax
import jax.numpy as jnp
from jax import lax
from jax.experimental import pallas as pl
from jax.experimental.pallas import tpu as pltpu
from jax.experimental.pallas import tpu_sc as plsc

C = 21
HW = 512 * 512
B = 4
K = 1024
HSIZE = 2 * C * K
NTEC = 32
PER_TEC = HW // NTEC
S = 1024
CPB = PER_TEC // S
NCHUNK = B * CPB
FGOFF = C * K


def _sc_body(preds_hbm, labels_hbm, out_hbm, buf, lbuf, hist, sem0, sem1):
    wid = lax.axis_index("c") * 16 + lax.axis_index("s")
    sems = (sem0, sem1)

    def zero_body(i, _):
        hist[pl.ds(i * 16, 16)] = jnp.zeros((16,), jnp.int32)
        return 0

    lax.fori_loop(0, HSIZE // 16, zero_body, 0)

    def window(t):
        b = t // CPB
        gw = wid * CPB + (t % CPB)
        r0 = (gw // 4) * 8
        c0 = (gw % 4) * 128
        return b, r0, c0

    def issue(t, slot):
        b, r0, c0 = window(t)
        pltpu.async_copy(
            preds_hbm.at[b, :, pl.ds(r0, 8), pl.ds(c0, 128)],
            buf.at[slot],
            sems[slot],
        )
        pltpu.async_copy(
            labels_hbm.at[b, pl.ds(r0, 8), pl.ds(c0, 128)],
            lbuf.at[slot],
            sems[slot],
        )

    def drain(slot):
        pltpu.make_async_copy(
            preds_hbm.at[0, :, pl.ds(0, 8), pl.ds(0, 128)], buf.at[slot], sems[slot]
        ).wait()
        pltpu.make_async_copy(
            labels_hbm.at[0, pl.ds(0, 8), pl.ds(0, 128)], lbuf.at[slot], sems[slot]
        ).wait()

    def compute(slot):
        def group(i, _):
            r = i // (128 // 16)
            s0 = (i % (128 // 16)) * 16
            es = []
            tot = jnp.zeros((16,), jnp.float32)
            for c in range(C):
                ex = jnp.exp(buf[slot, c, r, pl.ds(s0, 16)])
                es.append(ex)
                tot = tot + ex
            inv = jnp.float32(K) / tot
            lbl = lbuf[slot, r, pl.ds(s0, 16)]
            for c in range(C):
                t = es[c] * inv
                bi0 = jnp.minimum(t.astype(jnp.int32), K - 1)
                fgb = lbl == c
                idx = c * K + jnp.where(fgb, (FGOFF + K - 1) - bi0, bi0)
                plsc.addupdate_scatter(hist, [idx], jnp.ones((16,), jnp.int32))
            return 0

        lax.fori_loop(0, S // 16, group, 0)

    issue(jnp.int32(0), 0)

    def chunk_pair(j, _):
        t0 = j * 2
        issue(t0 + 1, 1)
        drain(0)
        compute(0)

        @pl.when(t0 + 2 < NCHUNK)
        def _():
            issue(t0 + 2, 0)

        drain(1)
        compute(1)
        return 0

    lax.fori_loop(0, NCHUNK // 2, chunk_pair, 0)

    pltpu.sync_copy(hist, out_hbm.at[pl.ds(wid * HSIZE, HSIZE)])


@functools.cache
def _sc_hist():
    return functools.partial(
        pl.kernel,
        out_type=jax.ShapeDtypeStruct((NTEC * HSIZE,), jnp.int32),
        mesh=plsc.VectorSubcoreMesh(core_axis_name="c", subcore_axis_name="s"),
        compiler_params=pltpu.CompilerParams(needs_layout_passes=False),
        scratch_types=[
            pltpu.VMEM((2, C, 8, 128), jnp.float32),
            pltpu.VMEM((2, 8, 128), jnp.int32),
            pltpu.VMEM((HSIZE,), jnp.int32),
            pltpu.SemaphoreType.DMA,
            pltpu.SemaphoreType.DMA,
        ],
    )(_sc_body)


def _tc_body(hist_ref, out_ref):
    h = hist_ref[...].astype(jnp.float32)
    acc = jnp.sum(h, axis=0)
    hb = acc[0:C, :]
    hf = acc[C : 2 * C, :]

    row = lax.broadcasted_iota(jnp.int32, (K, K), 0)
    col = lax.broadcasted_iota(jnp.int32, (K, K), 1)
    ge = (row >= col).astype(jnp.float32)
    f_incl = jax.lax.dot(hf, ge, preferred_element_type=jnp.float32)
    b_incl = jax.lax.dot(hb, ge, preferred_element_type=jnp.float32)
    f_excl = f_incl - hf
    b_excl = b_incl - hb

    g = jnp.sum(hf, axis=1, keepdims=True)

    def jac(f, bb):
        return 1.0 - (g - f) / jnp.maximum(g + bb, 1.0)

    dj = jac(f_incl, b_incl) - jac(f_excl, b_excl)
    ebar = (lax.broadcasted_iota(jnp.int32, (1, K), 1).astype(jnp.float32) + 0.5) * jnp.float32(1.0 / K)
    loss_c = jnp.sum(ebar * dj, axis=1, keepdims=True)
    present = (g > 0).astype(jnp.float32)
    num = jnp.sum(loss_c * present)
    den = jnp.maximum(jnp.sum(present), 1.0)
    out_ref[...] = jnp.broadcast_to(num / den, (1, 1))


def _tc_finish(hist3):
    return pl.pallas_call(
        _tc_body,
        out_shape=jax.ShapeDtypeStruct((1, 1), jnp.float32),
    )(hist3)


def kernel(preds, target):
    x = preds[0]
    lbl = target
    hist = _sc_hist()(x, lbl)
    hist3 = hist.reshape(NTEC, 2 * C, K)
    out = _tc_finish(hist3)
    return out[0, 0]


# speedup vs baseline: 201.4963x; 1.1693x over previous
import functools

import jax
import jax.numpy as jnp
from jax import lax
from jax.experimental import pallas as pl
from jax.experimental.pallas import tpu as pltpu
from jax.experimental.pallas import tpu_sc as plsc

C = 21
HW = 512 * 512
B = 4
K = 1024
HSIZE = 2 * C * K
NTEC = 32
PER_TEC = HW // NTEC
S = 1024
CPB = PER_TEC // S
NCHUNK = B * CPB
FGOFF = C * K


def _sc_body(preds_hbm, labels_hbm, out_hbm, buf, lbuf, hist, sem0, sem1):
    wid = lax.axis_index("c") * 16 + lax.axis_index("s")
    sems = (sem0, sem1)

    def zero_body(i, _):
        hist[pl.ds(i * 16, 16)] = jnp.zeros((16,), jnp.int32)
        return 0

    lax.fori_loop(0, HSIZE // 16, zero_body, 0)

    def window(t):
        b = t // CPB
        gw = wid * CPB + (t % CPB)
        r0 = (gw // 4) * 8
        c0 = (gw % 4) * 128
        return b, r0, c0

    def issue(t, slot):
        b, r0, c0 = window(t)
        pltpu.async_copy(
            preds_hbm.at[b, :, pl.ds(r0, 8), pl.ds(c0, 128)],
            buf.at[slot],
            sems[slot],
        )
        pltpu.async_copy(
            labels_hbm.at[b, pl.ds(r0, 8), pl.ds(c0, 128)],
            lbuf.at[slot],
            sems[slot],
        )

    def drain(slot):
        pltpu.make_async_copy(
            preds_hbm.at[0, :, pl.ds(0, 8), pl.ds(0, 128)], buf.at[slot], sems[slot]
        ).wait()
        pltpu.make_async_copy(
            labels_hbm.at[0, pl.ds(0, 8), pl.ds(0, 128)], lbuf.at[slot], sems[slot]
        ).wait()

    def compute(slot):
        ones = jnp.ones((16,), jnp.int32)

        def group(i, _):
            r = i // 4
            base = (i % 4) * 32
            subs = []
            for u in range(2):
                s0 = base + u * 16
                es = []
                tot = jnp.zeros((16,), jnp.float32)
                for c in range(C):
                    ex = jnp.exp(buf[slot, c, r, pl.ds(s0, 16)])
                    es.append(ex)
                    tot = tot + ex
                inv = jnp.float32(K) / tot
                lbl = lbuf[slot, r, pl.ds(s0, 16)]
                subs.append((es, inv, lbl))
            for es, inv, lbl in subs:
                for c in range(C):
                    t = es[c] * inv
                    bi0 = jnp.minimum(t.astype(jnp.int32), K - 1)
                    fgb = lbl == c
                    idx = c * K + jnp.where(fgb, (FGOFF + K - 1) - bi0, bi0)
                    plsc.addupdate_scatter(hist, [idx], ones)
            return 0

        lax.fori_loop(0, S // 32, group, 0)

    issue(jnp.int32(0), 0)

    def chunk_pair(j, _):
        t0 = j * 2
        issue(t0 + 1, 1)
        drain(0)
        compute(0)

        @pl.when(t0 + 2 < NCHUNK)
        def _():
            issue(t0 + 2, 0)

        drain(1)
        compute(1)
        return 0

    lax.fori_loop(0, NCHUNK // 2, chunk_pair, 0)

    pltpu.sync_copy(hist, out_hbm.at[pl.ds(wid * HSIZE, HSIZE)])


@functools.cache
def _sc_hist():
    return functools.partial(
        pl.kernel,
        out_type=jax.ShapeDtypeStruct((NTEC * HSIZE,), jnp.int32),
        mesh=plsc.VectorSubcoreMesh(core_axis_name="c", subcore_axis_name="s"),
        compiler_params=pltpu.CompilerParams(needs_layout_passes=False),
        scratch_types=[
            pltpu.VMEM((2, C, 8, 128), jnp.float32),
            pltpu.VMEM((2, 8, 128), jnp.int32),
            pltpu.VMEM((HSIZE,), jnp.int32),
            pltpu.SemaphoreType.DMA,
            pltpu.SemaphoreType.DMA,
        ],
    )(_sc_body)


def _tc_body(hist_ref, out_ref):
    h = hist_ref[...].astype(jnp.float32)
    acc = jnp.sum(h, axis=0)
    hb = acc[0:C, :]
    hf = acc[C : 2 * C, :]

    row = lax.broadcasted_iota(jnp.int32, (K, K), 0)
    col = lax.broadcasted_iota(jnp.int32, (K, K), 1)
    ge = (row >= col).astype(jnp.float32)
    f_incl = jax.lax.dot(hf, ge, preferred_element_type=jnp.float32)
    b_incl = jax.lax.dot(hb, ge, preferred_element_type=jnp.float32)
    f_excl = f_incl - hf
    b_excl = b_incl - hb

    g = jnp.sum(hf, axis=1, keepdims=True)

    def jac(f, bb):
        return 1.0 - (g - f) / jnp.maximum(g + bb, 1.0)

    dj = jac(f_incl, b_incl) - jac(f_excl, b_excl)
    ebar = (lax.broadcasted_iota(jnp.int32, (1, K), 1).astype(jnp.float32) + 0.5) * jnp.float32(1.0 / K)
    loss_c = jnp.sum(ebar * dj, axis=1, keepdims=True)
    present = (g > 0).astype(jnp.float32)
    num = jnp.sum(loss_c * present)
    den = jnp.maximum(jnp.sum(present), 1.0)
    out_ref[...] = jnp.broadcast_to(num / den, (1, 1))


def _tc_finish(hist3):
    return pl.pallas_call(
        _tc_body,
        out_shape=jax.ShapeDtypeStruct((1, 1), jnp.float32),
    )(hist3)


def kernel(preds, target):
    x = preds[0]
    lbl = target
    hist = _sc_hist()(x, lbl)
    hist3 = hist.reshape(NTEC, 2 * C, K)
    out = _tc_finish(hist3)
    return out[0, 0]
